# initial kernel scaffold (unmeasured)
import jax
import jax.numpy as jnp
from jax import lax
from jax.experimental import pallas as pl
from jax.experimental.pallas import tpu as pltpu

ROWS = 4096
COLS = 1024
CHUNK = 256
MAX_CHUNKS = ROWS // CHUNK + 1


def kernel(x, dest):
    my_x = lax.axis_index("x")

    keep = (dest == my_x).astype(jnp.int32)
    n_keep = jnp.sum(keep)
    order = jnp.argsort(keep, stable=True)
    buf = x.astype(jnp.bfloat16)[order]
    scal = n_keep.reshape(1).astype(jnp.int32)

    def body(scal_ref, buf_ref, out_ref, send_sems, recv_sems):
        mx = lax.axis_index("x")
        my = lax.axis_index("y")
        mz = lax.axis_index("z")
        peer = (1 - mx, my, mz)

        nk = scal_ref[0]
        ns = ROWS - nk
        dst_base = mx * nk
        recv_base = (1 - mx) * nk
        keep_base = mx * ns

        n_chunks = (ns + CHUNK - 1) // CHUNK
        k_chunks = (nk + CHUNK - 1) // CHUNK

        barrier_sem = pltpu.get_barrier_semaphore()
        pl.semaphore_signal(
            barrier_sem, inc=1, device_id=peer,
            device_id_type=pl.DeviceIdType.MESH,
        )
        pl.semaphore_wait(barrier_sem, 1)

        def chunk_rdma(i, off, dst_off):
            return pltpu.make_async_remote_copy(
                src_ref=buf_ref.at[pl.ds(off, CHUNK), :],
                dst_ref=out_ref.at[pl.ds(dst_off, CHUNK), :],
                send_sem=send_sems.at[i],
                recv_sem=recv_sems.at[i],
                device_id=peer,
                device_id_type=pl.DeviceIdType.MESH,
            )

        def send_off(i):
            return jnp.maximum(0, jnp.minimum(i * CHUNK, ns - CHUNK))

        for i in range(MAX_CHUNKS):
            off = send_off(i)

            @pl.when(i < n_chunks)
            def _(i=i, off=off):
                chunk_rdma(i, off, dst_base + off).start()

        for j in range(MAX_CHUNKS):
            off = jnp.maximum(0, jnp.minimum(j * CHUNK, nk - CHUNK))

            @pl.when(j < k_chunks)
            def _(j=j, off=off):
                out_ref[pl.ds(keep_base + off, CHUNK), :] = buf_ref[
                    pl.ds(ns + off, CHUNK), :
                ]

        for i in range(MAX_CHUNKS):
            off = send_off(i)

            @pl.when(i < n_chunks)
            def _(i=i, off=off):
                chunk_rdma(i, off, recv_base + off).wait_recv()

        for i in range(MAX_CHUNKS):
            off = send_off(i)

            @pl.when(i < n_chunks)
            def _(i=i, off=off):
                chunk_rdma(i, off, dst_base + off).wait_send()

    return pl.pallas_call(
        body,
        out_shape=jax.ShapeDtypeStruct((ROWS, COLS), jnp.bfloat16),
        in_specs=[
            pl.BlockSpec(memory_space=pltpu.SMEM),
            pl.BlockSpec(memory_space=pltpu.VMEM),
        ],
        out_specs=pl.BlockSpec(memory_space=pltpu.VMEM),
        scratch_shapes=[
            pltpu.SemaphoreType.DMA((MAX_CHUNKS,)),
            pltpu.SemaphoreType.DMA((MAX_CHUNKS,)),
        ],
        compiler_params=pltpu.CompilerParams(collective_id=0),
    )(scal, buf)


# baseline (device time: 181546 ns/iter reference)
import jax
import jax.numpy as jnp
from jax import lax
from jax.experimental import pallas as pl
from jax.experimental.pallas import tpu as pltpu

ROWS = 4096
COLS = 1024
CHUNK = 256
MAX_CHUNKS = ROWS // CHUNK


def kernel(x, dest):
    my_x = lax.axis_index("x")

    keep = (dest == my_x).astype(jnp.int32)
    n_keep = jnp.sum(keep)
    n_send = ROWS - n_keep
    order = jnp.argsort(keep, stable=True)
    buf = x.astype(jnp.bfloat16)[order]
    scal = n_send.reshape(1).astype(jnp.int32)

    def body(scal_ref, buf_ref, recv_ref, send_sems, recv_sems):
        mx = lax.axis_index("x")
        my = lax.axis_index("y")
        mz = lax.axis_index("z")
        peer = (1 - mx, my, mz)

        ns = scal_ref[0]
        n_chunks = (ns + CHUNK - 1) // CHUNK

        barrier_sem = pltpu.get_barrier_semaphore()
        pl.semaphore_signal(
            barrier_sem, inc=1, device_id=peer,
            device_id_type=pl.DeviceIdType.MESH,
        )
        pl.semaphore_wait(barrier_sem, 1)

        def chunk_rdma(i):
            return pltpu.make_async_remote_copy(
                src_ref=buf_ref.at[pl.ds(i * CHUNK, CHUNK), :],
                dst_ref=recv_ref.at[pl.ds(i * CHUNK, CHUNK), :],
                send_sem=send_sems.at[i],
                recv_sem=recv_sems.at[i],
                device_id=peer,
                device_id_type=pl.DeviceIdType.MESH,
            )

        for i in range(MAX_CHUNKS):
            @pl.when(i < n_chunks)
            def _(i=i):
                chunk_rdma(i).start()

        for i in range(MAX_CHUNKS):
            @pl.when(i < n_chunks)
            def _(i=i):
                chunk_rdma(i).wait_recv()

        for i in range(MAX_CHUNKS):
            @pl.when(i < n_chunks)
            def _(i=i):
                chunk_rdma(i).wait_send()

    recv = pl.pallas_call(
        body,
        out_shape=jax.ShapeDtypeStruct((ROWS, COLS), jnp.bfloat16),
        in_specs=[
            pl.BlockSpec(memory_space=pltpu.SMEM),
            pl.BlockSpec(memory_space=pltpu.VMEM),
        ],
        out_specs=pl.BlockSpec(memory_space=pltpu.VMEM),
        scratch_shapes=[
            pltpu.SemaphoreType.DMA((MAX_CHUNKS,)),
            pltpu.SemaphoreType.DMA((MAX_CHUNKS,)),
        ],
        compiler_params=pltpu.CompilerParams(collective_id=0),
    )(scal, buf)

    keep_base = my_x * n_send
    recv_base = (1 - my_x) * n_keep
    i = jnp.arange(ROWS)
    in_keep = (i >= keep_base) & (i < keep_base + n_keep)
    idx = jnp.where(in_keep, n_send + i - keep_base, ROWS + i - recv_base)
    combined = jnp.concatenate([buf, recv], axis=0)
    return combined[idx]


# device time: 109667 ns/iter; 1.6554x vs baseline; 1.6554x over previous
import jax
import jax.numpy as jnp
from jax import lax
from jax.experimental import pallas as pl
from jax.experimental.pallas import tpu as pltpu

ROWS = 4096
COLS = 1024
CHUNK = 256
N_CHUNKS = ROWS // CHUNK


def kernel(x, dest):
    my_x = lax.axis_index("x")

    keep = (dest == my_x).astype(jnp.int32)
    n_keep = jnp.sum(keep)
    n_send = ROWS - n_keep
    order = jnp.argsort(keep, stable=True).astype(jnp.int32).reshape(ROWS, 1)
    xb = x.astype(jnp.bfloat16)
    scal = n_send.reshape(1).astype(jnp.int32)

    def body(scal_ref, ord_ref, xb_ref, out_ref, buf_ref, recv_ref,
             send_sems, recv_sems):
        mx = lax.axis_index("x")
        my = lax.axis_index("y")
        mz = lax.axis_index("z")
        peer = (1 - mx, my, mz)

        ns = scal_ref[0]
        nk = ROWS - ns
        send_chunks = (ns + CHUNK - 1) // CHUNK

        barrier_sem = pltpu.get_barrier_semaphore()
        pl.semaphore_signal(
            barrier_sem, inc=1, device_id=peer,
            device_id_type=pl.DeviceIdType.MESH,
        )
        pl.semaphore_wait(barrier_sem, 1)

        def chunk_rdma(i):
            return pltpu.make_async_remote_copy(
                src_ref=buf_ref.at[pl.ds(i * CHUNK, CHUNK), :],
                dst_ref=recv_ref.at[pl.ds(i * CHUNK, CHUNK), :],
                send_sem=send_sems.at[i],
                recv_sem=recv_sems.at[i],
                device_id=peer,
                device_id_type=pl.DeviceIdType.MESH,
            )

        KC = 1024
        for c in range(N_CHUNKS):
            ords = ord_ref[pl.ds(c * CHUNK, CHUNK), :]
            rows = jnp.zeros((CHUNK, COLS), jnp.float32)
            for k in range(ROWS // KC):
                col = lax.broadcasted_iota(jnp.int32, (CHUNK, KC), 1) + k * KC
                p = (ords == col).astype(jnp.bfloat16)
                rows = rows + lax.dot_general(
                    p, xb_ref[pl.ds(k * KC, KC), :], (((1,), (0,)), ((), ())),
                    preferred_element_type=jnp.float32,
                )
            buf_ref[pl.ds(c * CHUNK, CHUNK), :] = rows.astype(jnp.bfloat16)

            @pl.when(c < send_chunks)
            def _(c=c):
                chunk_rdma(c).start()

        for c in range(N_CHUNKS):
            @pl.when(c < send_chunks)
            def _(c=c):
                chunk_rdma(c).wait_recv()

        for c in range(N_CHUNKS):
            @pl.when(c < send_chunks)
            def _(c=c):
                chunk_rdma(c).wait_send()

        keep_base = mx * ns
        recv_base = (1 - mx) * nk
        keep_shift = (keep_base - ns) % ROWS
        row = lax.broadcasted_iota(jnp.int32, (ROWS, 1), 0)
        in_keep = (row >= keep_base) & (row < keep_base + nk)
        SLAB = 256
        for s in range(COLS // SLAB):
            cols = pl.ds(s * SLAB, SLAB)
            rolled_keep = pltpu.roll(buf_ref[:, cols], keep_shift, 0)
            rolled_recv = pltpu.roll(recv_ref[:, cols], recv_base, 0)
            out_ref[:, cols] = jnp.where(in_keep, rolled_keep, rolled_recv)

    return pl.pallas_call(
        body,
        out_shape=jax.ShapeDtypeStruct((ROWS, COLS), jnp.bfloat16),
        in_specs=[
            pl.BlockSpec(memory_space=pltpu.SMEM),
            pl.BlockSpec(memory_space=pltpu.VMEM),
            pl.BlockSpec(memory_space=pltpu.VMEM),
        ],
        out_specs=pl.BlockSpec(memory_space=pltpu.VMEM),
        scratch_shapes=[
            pltpu.VMEM((ROWS, COLS), jnp.bfloat16),
            pltpu.VMEM((ROWS, COLS), jnp.bfloat16),
            pltpu.SemaphoreType.DMA((N_CHUNKS,)),
            pltpu.SemaphoreType.DMA((N_CHUNKS,)),
        ],
        compiler_params=pltpu.CompilerParams(
            collective_id=0, vmem_limit_bytes=100 * 1024 * 1024
        ),
    )(scal, order, xb)
